# baseline (speedup 1.0000x reference)
import jax
import jax.numpy as jnp
from jax.experimental import pallas as pl
from jax.experimental.pallas import tpu as pltpu

D = 1024
F = 1408
NE = 16
NSH = 2
TT = 1024
NF = 11
FB = F // NF
HI = jax.lax.Precision.HIGHEST


def _dgt(a, b, prec=None):
    return jax.lax.dot_general(a, b, (((1,), (1,)), ((), ())),
                               precision=prec,
                               preferred_element_type=jnp.float32)


def _silu(v):
    return v * jax.nn.sigmoid(v)


def _routed_body(x_ref, gw_ref, w1_ref, w3_ref, w2_ref, o_ref, g_scr):
    e = pl.program_id(1)
    f = pl.program_id(2)
    x = x_ref[...]

    @pl.when((e == 0) & (f == 0))
    def _():
        logits = _dgt(x, gw_ref[...])
        m = jnp.max(logits, axis=1, keepdims=True)
        p = jnp.exp(logits - m)
        p = p / jnp.sum(p, axis=1, keepdims=True)
        lane = jax.lax.broadcasted_iota(jnp.int32, (TT, NE), 1)
        m1 = jnp.max(p, axis=1, keepdims=True)
        i1 = jnp.min(jnp.where(p == m1, lane, NE), axis=1, keepdims=True)
        p2 = jnp.where(lane == i1, -jnp.inf, p)
        m2 = jnp.max(p2, axis=1, keepdims=True)
        i2 = jnp.min(jnp.where(p2 == m2, lane, NE), axis=1, keepdims=True)
        g_scr[...] = (jnp.where(lane == i1, m1, 0.0)
                      + jnp.where(lane == i2, m2, 0.0))
        o_ref[...] = jnp.zeros_like(o_ref)

    lane = jax.lax.broadcasted_iota(jnp.int32, (TT, NE), 1)
    gate = jnp.sum(jnp.where(lane == e, g_scr[...], 0.0), axis=1,
                   keepdims=True)
    h = _silu(_dgt(x, w1_ref[0])) * _dgt(x, w3_ref[0])
    o_ref[...] += _dgt(h, w2_ref[0]) * gate


def _shared_body(x_ref, y_ref, s1_ref, s3_ref, s2_ref, o_ref):
    c = pl.program_id(1)
    f = pl.program_id(2)
    x = x_ref[...]

    @pl.when((c == 0) & (f == 0))
    def _():
        o_ref[...] = y_ref[...]

    h = _silu(_dgt(x, s1_ref[0])) * _dgt(x, s3_ref[0])
    o_ref[...] += _dgt(h, s2_ref[0])


def kernel(x, gate_w, w1, w2, w3, sw1, sw2, sw3):
    orig_shape = x.shape
    xf = x.reshape(-1, D)
    n = xf.shape[0]
    nt = n // TT

    y = pl.pallas_call(
        _routed_body,
        grid=(nt, NE, NF),
        in_specs=[
            pl.BlockSpec((TT, D), lambda t, e, f: (t, 0)),
            pl.BlockSpec((NE, D), lambda t, e, f: (0, 0)),
            pl.BlockSpec((1, FB, D), lambda t, e, f: (e, f, 0)),
            pl.BlockSpec((1, FB, D), lambda t, e, f: (e, f, 0)),
            pl.BlockSpec((1, D, FB), lambda t, e, f: (e, 0, f)),
        ],
        out_specs=pl.BlockSpec((TT, D), lambda t, e, f: (t, 0)),
        out_shape=jax.ShapeDtypeStruct((n, D), jnp.float32),
        scratch_shapes=[pltpu.VMEM((TT, NE), jnp.float32)],
    )(xf, gate_w, w1, w3, w2)

    s1r = sw1.reshape(NSH, F, D)
    s3r = sw3.reshape(NSH, F, D)
    s2r = sw2.reshape(D, NSH, F).transpose(1, 0, 2)
    out = pl.pallas_call(
        _shared_body,
        grid=(nt, NSH, NF),
        in_specs=[
            pl.BlockSpec((TT, D), lambda t, c, f: (t, 0)),
            pl.BlockSpec((TT, D), lambda t, c, f: (t, 0)),
            pl.BlockSpec((1, FB, D), lambda t, c, f: (c, f, 0)),
            pl.BlockSpec((1, FB, D), lambda t, c, f: (c, f, 0)),
            pl.BlockSpec((1, D, FB), lambda t, c, f: (c, 0, f)),
        ],
        out_specs=pl.BlockSpec((TT, D), lambda t, c, f: (t, 0)),
        out_shape=jax.ShapeDtypeStruct((n, D), jnp.float32),
    )(xf, y, s1r, s3r, s2r)

    return out.reshape(orig_shape)


# speedup vs baseline: 2.1552x; 2.1552x over previous
import jax
import jax.numpy as jnp
from jax.experimental import pallas as pl
from jax.experimental.pallas import tpu as pltpu

D = 1024
F = 1408
NE = 16
NSH = 2
TOPK = 2
TT = 1024
TM = 128
NF = 11
FB = F // NF


def _dgt(a, b):
    return jax.lax.dot_general(a, b, (((1,), (1,)), ((), ())),
                               preferred_element_type=jnp.float32)


def _silu(v):
    return v * jax.nn.sigmoid(v)


def _gate_body(x_ref, gw_ref, idx_ref, wgt_ref):
    logits = _dgt(x_ref[...], gw_ref[...])
    m = jnp.max(logits, axis=1, keepdims=True)
    p = jnp.exp(logits - m)
    p = p / jnp.sum(p, axis=1, keepdims=True)
    lane = jax.lax.broadcasted_iota(jnp.int32, (TT, NE), 1)
    m1 = jnp.max(p, axis=1, keepdims=True)
    i1 = jnp.min(jnp.where(p == m1, lane, NE), axis=1, keepdims=True)
    p2 = jnp.where(lane == i1, -jnp.inf, p)
    m2 = jnp.max(p2, axis=1, keepdims=True)
    i2 = jnp.min(jnp.where(p2 == m2, lane, NE), axis=1, keepdims=True)
    idx_ref[...] = jnp.concatenate([i1, i2], axis=1)
    wgt_ref[...] = jnp.concatenate([m1, m2], axis=1)


def _gmm_body(te_ref, xs_ref, g_ref, w1_ref, w3_ref, w2_ref, o_ref):
    x = xs_ref[...]
    h = _silu(_dgt(x, w1_ref[0])) * _dgt(x, w3_ref[0])
    o_ref[...] = _dgt(h, w2_ref[0]) * g_ref[...]


def _shared_body(x_ref, y_ref, s1_ref, s3_ref, s2_ref, o_ref):
    c = pl.program_id(1)
    f = pl.program_id(2)
    x = x_ref[...]

    @pl.when((c == 0) & (f == 0))
    def _():
        o_ref[...] = y_ref[...]

    h = _silu(_dgt(x, s1_ref[0])) * _dgt(x, s3_ref[0])
    o_ref[...] += _dgt(h, s2_ref[0])


def kernel(x, gate_w, w1, w2, w3, sw1, sw2, sw3):
    orig_shape = x.shape
    xf = x.reshape(-1, D)
    n = xf.shape[0]
    nt = n // TT
    a = n * TOPK
    pn = a + NE * TM
    nrt = pn // TM

    idx, wgt = pl.pallas_call(
        _gate_body,
        grid=(nt,),
        in_specs=[
            pl.BlockSpec((TT, D), lambda t: (t, 0)),
            pl.BlockSpec((NE, D), lambda t: (0, 0)),
        ],
        out_specs=[
            pl.BlockSpec((TT, TOPK), lambda t: (t, 0)),
            pl.BlockSpec((TT, TOPK), lambda t: (t, 0)),
        ],
        out_shape=[
            jax.ShapeDtypeStruct((n, TOPK), jnp.int32),
            jax.ShapeDtypeStruct((n, TOPK), jnp.float32),
        ],
    )(xf, gate_w)

    e_flat = idx.reshape(-1)
    w_flat = wgt.reshape(-1)
    t_flat = jnp.arange(a, dtype=jnp.int32) // TOPK
    counts = jnp.zeros((NE,), jnp.int32).at[e_flat].add(1)
    cnt_pad = ((counts + TM - 1) // TM) * TM
    start = jnp.concatenate([jnp.zeros((1,), jnp.int32),
                             jnp.cumsum(cnt_pad, dtype=jnp.int32)])
    start_unp = jnp.concatenate([jnp.zeros((1,), jnp.int32),
                                 jnp.cumsum(counts, dtype=jnp.int32)])[:NE]
    order = jnp.argsort(e_flat, stable=True).astype(jnp.int32)
    e_sorted = e_flat[order]
    rank = jnp.arange(a, dtype=jnp.int32) - start_unp[e_sorted]
    p_sorted = start[:NE][e_sorted] + rank
    pos = jnp.zeros((a,), jnp.int32).at[order].set(p_sorted)
    row_token = jnp.zeros((pn,), jnp.int32).at[p_sorted].set(t_flat[order])
    row_gate = jnp.zeros((pn, 1), jnp.float32).at[p_sorted, 0].set(
        w_flat[order])
    tile_expert = jnp.clip(
        jnp.searchsorted(start, jnp.arange(nrt, dtype=jnp.int32) * TM,
                         side='right').astype(jnp.int32) - 1, 0, NE - 1)

    xs = xf[row_token]

    eout = pl.pallas_call(
        _gmm_body,
        grid_spec=pltpu.PrefetchScalarGridSpec(
            num_scalar_prefetch=1,
            grid=(nrt,),
            in_specs=[
                pl.BlockSpec((TM, D), lambda i, te: (i, 0)),
                pl.BlockSpec((TM, 1), lambda i, te: (i, 0)),
                pl.BlockSpec((1, F, D), lambda i, te: (te[i], 0, 0)),
                pl.BlockSpec((1, F, D), lambda i, te: (te[i], 0, 0)),
                pl.BlockSpec((1, D, F), lambda i, te: (te[i], 0, 0)),
            ],
            out_specs=pl.BlockSpec((TM, D), lambda i, te: (i, 0)),
        ),
        out_shape=jax.ShapeDtypeStruct((pn, D), jnp.float32),
    )(tile_expert, xs, row_gate, w1, w3, w2)

    y_r = eout[pos[0::2]] + eout[pos[1::2]]

    s1r = sw1.reshape(NSH, F, D)
    s3r = sw3.reshape(NSH, F, D)
    s2r = sw2.reshape(D, NSH, F).transpose(1, 0, 2)
    out = pl.pallas_call(
        _shared_body,
        grid=(nt, NSH, NF),
        in_specs=[
            pl.BlockSpec((TT, D), lambda t, c, f: (t, 0)),
            pl.BlockSpec((TT, D), lambda t, c, f: (t, 0)),
            pl.BlockSpec((1, FB, D), lambda t, c, f: (c, f, 0)),
            pl.BlockSpec((1, FB, D), lambda t, c, f: (c, f, 0)),
            pl.BlockSpec((1, D, FB), lambda t, c, f: (c, 0, f)),
        ],
        out_specs=pl.BlockSpec((TT, D), lambda t, c, f: (t, 0)),
        out_shape=jax.ShapeDtypeStruct((n, D), jnp.float32),
    )(xf, y_r, s1r, s3r, s2r)

    return out.reshape(orig_shape)


# speedup vs baseline: 2.5998x; 1.2063x over previous
import functools

import jax
import jax.numpy as jnp
from jax import lax
from jax.experimental import pallas as pl
from jax.experimental.pallas import tpu as pltpu
from jax.experimental.pallas import tpu_sc as plsc

D = 1024
F = 1408
NE = 16
NSH = 2
TOPK = 2
TT = 1024
TM = 128
NF = 11
FB = F // NF
N = 4096
A = N * TOPK
PN = A + NE * TM
NRT = PN // TM
NW = 32
NWR = 16


def _dgt(a, b):
    return jax.lax.dot_general(a, b, (((1,), (1,)), ((), ())),
                               preferred_element_type=jnp.float32)


def _silu(v):
    return v * jax.nn.sigmoid(v)


def _gate_body(x_ref, gw_ref, idx_ref, wgt_ref):
    logits = _dgt(x_ref[...], gw_ref[...])
    m = jnp.max(logits, axis=1, keepdims=True)
    p = jnp.exp(logits - m)
    p = p / jnp.sum(p, axis=1, keepdims=True)
    lane = jax.lax.broadcasted_iota(jnp.int32, (TT, NE), 1)
    m1 = jnp.max(p, axis=1, keepdims=True)
    i1 = jnp.min(jnp.where(p == m1, lane, NE), axis=1, keepdims=True)
    p2 = jnp.where(lane == i1, -jnp.inf, p)
    m2 = jnp.max(p2, axis=1, keepdims=True)
    i2 = jnp.min(jnp.where(p2 == m2, lane, NE), axis=1, keepdims=True)
    idx_ref[...] = jnp.concatenate([i1, i2], axis=1)
    wgt_ref[...] = jnp.concatenate([m1, m2], axis=1)


def _gmm_body(te_ref, xs_ref, w1_ref, w3_ref, w2_ref, o_ref):
    x = xs_ref[...]
    h = _silu(_dgt(x, w1_ref[0])) * _dgt(x, w3_ref[0])
    o_ref[...] = _dgt(h, w2_ref[0])


def _shared_body(x_ref, s1_ref, s3_ref, s2_ref, o_ref):
    c = pl.program_id(1)
    f = pl.program_id(2)
    x = x_ref[...]

    @pl.when((c == 0) & (f == 0))
    def _():
        o_ref[...] = jnp.zeros_like(o_ref)

    h = _silu(_dgt(x, s1_ref[0])) * _dgt(x, s3_ref[0])
    o_ref[...] += _dgt(h, s2_ref[0])


_I16 = lambda: lax.iota(jnp.int32, 16)


def _route_body(idx_hbm, pos_hbm, te_hbm,
                idxv, histv, hist_sh, histall, posall, posd0, posd1, tev):
    wid = lax.axis_index("s")
    apw = (N // NWR) * TOPK
    a0 = wid * apw
    z16 = jnp.zeros((16,), jnp.int32)
    one16 = jnp.full((16,), 1, jnp.int32)
    ioq = _I16()

    pltpu.sync_copy(idx_hbm.at[pl.ds(a0, apw)], idxv)

    hist = z16
    for blk in range(apw // 16):
        ev = idxv[pl.ds(blk * 16, 16)]
        for e in range(NE):
            pop = jnp.sum(jnp.where(ev == e, one16, z16))
            hist = hist + jnp.where(ioq == e, pop, z16)
    histv[...] = hist
    pltpu.sync_copy(histv, hist_sh.at[pl.ds(wid * 16, 16)])
    plsc.subcore_barrier()
    plsc.subcore_barrier()
    pltpu.sync_copy(hist_sh, histall)

    widv = jnp.full((16,), wid, jnp.int32)
    tot = z16
    mycum = z16
    for r in range(NWR):
        row = histall[pl.ds(r * 16, 16)]
        tot = tot + row
        mask = jnp.full((16,), r, jnp.int32) < widv
        mycum = mycum + jnp.where(mask, row, z16)
    cnt_pad = ((tot + (TM - 1)) >> 7) << 7
    start_incl = plsc.cumsum(cnt_pad)
    start_excl = start_incl - cnt_pad

    @pl.when(wid == 0)
    def _():
        for ch in range(NRT // 16):
            tb = (ioq + 16 * ch) * TM
            acc = z16
            for e in range(NE):
                se = jnp.sum(jnp.where(ioq == e, start_incl, z16))
                acc = acc + jnp.where(tb >= se, one16, z16)
            tev[pl.ds(ch * 16, 16)] = jnp.minimum(acc, NE - 1)
        pltpu.sync_copy(tev, te_hbm)

    cur = start_excl + mycum
    for blk in range(apw // 16):
        ev = idxv[pl.ds(blk * 16, 16)]
        posv = z16
        for e in range(NE):
            m = ev == e
            mi = jnp.where(m, one16, z16)
            pre = plsc.cumsum(mi)
            ce = jnp.sum(jnp.where(ioq == e, cur, z16))
            cnt = jnp.sum(mi)
            posv = posv + jnp.where(m, pre - 1 + ce, z16)
            cur = cur + jnp.where(ioq == e, cnt, z16)
        posall[pl.ds(blk * 16, 16)] = posv

    plsc.subcore_barrier()
    tpw = N // NWR
    for blk in range(tpw // 16):
        base = 32 * blk
        v0 = plsc.load_gather(posall, [ioq * 2 + base])
        v1 = plsc.load_gather(posall, [ioq * 2 + base + 1])
        posd0[pl.ds(blk * 16, 16)] = v0
        posd1[pl.ds(blk * 16, 16)] = v1
    pltpu.sync_copy(posd0, pos_hbm.at[0, pl.ds(wid * tpw, tpw)])
    pltpu.sync_copy(posd1, pos_hbm.at[1, pl.ds(wid * tpw, tpw)])


def _dispatch_body(xf_hbm, pos_hbm, xs_hbm, pv0, pv1, xbuf, semr, semw):
    wid = lax.axis_index("s") * 2 + lax.axis_index("c")
    tpw = N // NW
    t0 = wid * tpw
    nch = tpw // 16
    for c in range(nch):
        pltpu.sync_copy(pos_hbm.at[0, pl.ds(t0 + 16 * c, 16)], pv0.at[c])
        pltpu.sync_copy(pos_hbm.at[1, pl.ds(t0 + 16 * c, 16)], pv1.at[c])

    rd = pltpu.async_copy(xf_hbm.at[pl.ds(t0, 16)], xbuf.at[0], semr)
    for s in range(nch):
        rd.wait()
        if s + 1 < nch:
            rd = pltpu.async_copy(
                xf_hbm.at[pl.ds(t0 + 16 * (s + 1), 16)],
                xbuf.at[(s + 1) % 2], semr)
        w0 = pltpu.async_copy(xbuf.at[s % 2], xs_hbm.at[pv0.at[s]], semw)
        w1 = pltpu.async_copy(xbuf.at[s % 2], xs_hbm.at[pv1.at[s]], semw)
        w0.wait()
        w1.wait()


def _combine_body(eout_hbm, z_hbm, pos_hbm, wgt_hbm, out_hbm,
                  p0v, p1v, w0v, w1v, abuf, bbuf, zbuf, obuf, sem):
    wid = lax.axis_index("s") * 2 + lax.axis_index("c")
    tpw = N // NW
    t0 = wid * tpw
    ioq = _I16()
    zf16 = jnp.zeros((16,), jnp.float32)
    pltpu.sync_copy(wgt_hbm.at[0, pl.ds(t0, tpw)], w0v)
    pltpu.sync_copy(wgt_hbm.at[1, pl.ds(t0, tpw)], w1v)
    pltpu.sync_copy(pos_hbm.at[0, pl.ds(t0, tpw)], p0v)
    pltpu.sync_copy(pos_hbm.at[1, pl.ds(t0, tpw)], p1v)
    plsc.subcore_barrier()

    for b in range(tpw // 16):
        ia = p0v[pl.ds(b * 16, 16)]
        pltpu.async_copy(eout_hbm.at[ia], abuf, sem).wait()
        ib = p1v[pl.ds(b * 16, 16)]
        pltpu.async_copy(eout_hbm.at[ib], bbuf, sem).wait()
        pltpu.sync_copy(z_hbm.at[pl.ds(t0 + 16 * b, 16)], zbuf)
        w0row = w0v[pl.ds(b * 16, 16)]
        w1row = w1v[pl.ds(b * 16, 16)]

        for i in range(16):
            w0 = jnp.sum(jnp.where(ioq == i, w0row, zf16))
            w1 = jnp.sum(jnp.where(ioq == i, w1row, zf16))

            def ch_step(ch, _, i=i, w0=w0, w1=w1):
                s = pl.ds(ch * 16, 16)
                obuf[i, s] = (w0 * abuf[i, s] + w1 * bbuf[i, s]
                              + zbuf[i, s])
                return 0

            lax.fori_loop(0, D // 16, ch_step, 0, unroll=4)

        pltpu.sync_copy(obuf, out_hbm.at[pl.ds(t0 + 16 * b, 16)])


def kernel(x, gate_w, w1, w2, w3, sw1, sw2, sw3):
    orig_shape = x.shape
    xf = x.reshape(-1, D)
    n = xf.shape[0]
    nt = n // TT

    idx, wgt = pl.pallas_call(
        _gate_body,
        grid=(nt,),
        in_specs=[
            pl.BlockSpec((TT, D), lambda t: (t, 0)),
            pl.BlockSpec((NE, D), lambda t: (0, 0)),
        ],
        out_specs=[
            pl.BlockSpec((TT, TOPK), lambda t: (t, 0)),
            pl.BlockSpec((TT, TOPK), lambda t: (t, 0)),
        ],
        out_shape=[
            jax.ShapeDtypeStruct((n, TOPK), jnp.int32),
            jax.ShapeDtypeStruct((n, TOPK), jnp.float32),
        ],
    )(xf, gate_w)
    idx_flat = idx.reshape(-1)
    wgt_flat = wgt.reshape(-1)

    route_mesh = plsc.VectorSubcoreMesh(
        core_axis_name="c", subcore_axis_name="s", num_cores=1)
    pos, te = pl.kernel(
        _route_body,
        out_type=[
            jax.ShapeDtypeStruct((TOPK, N), jnp.int32),
            jax.ShapeDtypeStruct((NRT,), jnp.int32),
        ],
        mesh=route_mesh,
        compiler_params=pltpu.CompilerParams(needs_layout_passes=False),
        scratch_types=[
            pltpu.VMEM(((N // NWR) * TOPK,), jnp.int32),
            pltpu.VMEM((16,), jnp.int32),
            pltpu.VMEM_SHARED((NWR * 16,), jnp.int32),
            pltpu.VMEM((NWR * 16,), jnp.int32),
            pltpu.VMEM(((N // NWR) * TOPK,), jnp.int32),
            pltpu.VMEM((N // NWR,), jnp.int32),
            pltpu.VMEM((N // NWR,), jnp.int32),
            pltpu.VMEM((NRT,), jnp.int32),
        ],
    )(idx_flat)

    full_mesh = plsc.VectorSubcoreMesh(
        core_axis_name="c", subcore_axis_name="s")
    xs = pl.kernel(
        _dispatch_body,
        out_type=jax.ShapeDtypeStruct((PN, D), jnp.float32),
        mesh=full_mesh,
        compiler_params=pltpu.CompilerParams(needs_layout_passes=False),
        scratch_types=[
            pltpu.VMEM((N // NW // 16, 16), jnp.int32),
            pltpu.VMEM((N // NW // 16, 16), jnp.int32),
            pltpu.VMEM((2, 16, D), jnp.float32),
            pltpu.SemaphoreType.DMA,
            pltpu.SemaphoreType.DMA,
        ],
    )(xf, pos)

    eout = pl.pallas_call(
        _gmm_body,
        grid_spec=pltpu.PrefetchScalarGridSpec(
            num_scalar_prefetch=1,
            grid=(NRT,),
            in_specs=[
                pl.BlockSpec((TM, D), lambda i, te: (i, 0)),
                pl.BlockSpec((1, F, D), lambda i, te: (te[i], 0, 0)),
                pl.BlockSpec((1, F, D), lambda i, te: (te[i], 0, 0)),
                pl.BlockSpec((1, D, F), lambda i, te: (te[i], 0, 0)),
            ],
            out_specs=pl.BlockSpec((TM, D), lambda i, te: (i, 0)),
        ),
        out_shape=jax.ShapeDtypeStruct((PN, D), jnp.float32),
    )(te, xs, w1, w3, w2)

    s1r = sw1.reshape(NSH, F, D)
    s3r = sw3.reshape(NSH, F, D)
    s2r = sw2.reshape(D, NSH, F).transpose(1, 0, 2)
    z = pl.pallas_call(
        _shared_body,
        grid=(nt, NSH, NF),
        in_specs=[
            pl.BlockSpec((TT, D), lambda t, c, f: (t, 0)),
            pl.BlockSpec((1, FB, D), lambda t, c, f: (c, f, 0)),
            pl.BlockSpec((1, FB, D), lambda t, c, f: (c, f, 0)),
            pl.BlockSpec((1, D, FB), lambda t, c, f: (c, 0, f)),
        ],
        out_specs=pl.BlockSpec((TT, D), lambda t, c, f: (t, 0)),
        out_shape=jax.ShapeDtypeStruct((n, D), jnp.float32),
    )(xf, s1r, s3r, s2r)

    out = pl.kernel(
        _combine_body,
        out_type=jax.ShapeDtypeStruct((N, D), jnp.float32),
        mesh=full_mesh,
        compiler_params=pltpu.CompilerParams(needs_layout_passes=False),
        scratch_types=[
            pltpu.VMEM((N // NW,), jnp.int32),
            pltpu.VMEM((N // NW,), jnp.int32),
            pltpu.VMEM((N // NW,), jnp.float32),
            pltpu.VMEM((N // NW,), jnp.float32),
            pltpu.VMEM((16, D), jnp.float32),
            pltpu.VMEM((16, D), jnp.float32),
            pltpu.VMEM((16, D), jnp.float32),
            pltpu.VMEM((16, D), jnp.float32),
            pltpu.SemaphoreType.DMA,
        ],
    )(eout, z, pos, wgt.T)

    return out.reshape(orig_shape)
